# baseline (speedup 1.0000x reference)
import functools

import jax
import jax.numpy as jnp
from jax import lax
from jax.experimental import pallas as pl
from jax.experimental.pallas import tpu as pltpu
from jax.experimental.pallas import tpu_sc as plsc

N_ROWS = 1000000
K = 32
B = 16384
GRAN = 16
RPK = N_ROWS // GRAN

NC = 2
NS = 16
NW = NC * NS
BPW = B // NW
L = 16
C = 64
NCH = BPW // C
NG = C // L

_mesh = plsc.VectorSubcoreMesh(core_axis_name="c", subcore_axis_name="s")


@functools.partial(
    pl.kernel,
    out_type=jax.ShapeDtypeStruct((B,), jnp.float32),
    mesh=_mesh,
    scratch_types=[
        pltpu.VMEM((BPW,), jnp.int32),
        pltpu.VMEM((BPW,), jnp.int32),
        pltpu.VMEM((C * K,), jnp.int32),
        pltpu.VMEM((C * K,), jnp.int32),
        pltpu.VMEM((C * K, GRAN), jnp.float32),
        pltpu.VMEM((C * K, GRAN), jnp.float32),
        pltpu.VMEM((BPW,), jnp.float32),
        pltpu.SemaphoreType.DMA,
    ],
    compiler_params=pltpu.CompilerParams(needs_layout_passes=False,
                                         use_tc_tiling_on_sc=False),
)
def _svd_dot(u_idx_hbm, i_idx_hbm, u_flat_hbm, i_flat_hbm, out_hbm,
             uidx_v, iidx_v, urow_v, irow_v, ugath_v, igath_v, out_v, sem):
    wid = lax.axis_index("s") * NC + lax.axis_index("c")
    base = wid * BPW

    pltpu.sync_copy(u_idx_hbm.at[pl.ds(base, BPW)], uidx_v)
    pltpu.sync_copy(i_idx_hbm.at[pl.ds(base, BPW)], iidx_v)

    lane = lax.iota(jnp.int32, L)

    def chunk(c, carry):
        for g in range(NG):
            ur = uidx_v[pl.ds(c * C + g * L, L)]
            ir = iidx_v[pl.ds(c * C + g * L, L)]
            uhi = lax.shift_right_logical(ur, 4)
            ihi = lax.shift_right_logical(ir, 4)
            for k in range(K):
                urow_v[pl.ds(k * C + g * L, L)] = uhi + (k * RPK)
                irow_v[pl.ds(k * C + g * L, L)] = ihi + (k * RPK)

        cp_u = pltpu.async_copy(u_flat_hbm.at[urow_v], ugath_v, sem)
        cp_i = pltpu.async_copy(i_flat_hbm.at[irow_v], igath_v, sem)
        cp_u.wait()
        cp_i.wait()

        for g in range(NG):
            ur = uidx_v[pl.ds(c * C + g * L, L)]
            ir = iidx_v[pl.ds(c * C + g * L, L)]
            ulane = lax.bitwise_and(ur, GRAN - 1)
            ilane = lax.bitwise_and(ir, GRAN - 1)
            acc = jnp.zeros((L,), jnp.float32)
            for k in range(K):
                slot = lane + (k * C + g * L)
                uv = plsc.load_gather(ugath_v, [slot, ulane])
                iv = plsc.load_gather(igath_v, [slot, ilane])
                acc = acc + uv * iv
            out_v[pl.ds(c * C + g * L, L)] = acc
        return carry

    lax.fori_loop(0, NCH, chunk, 0)

    pltpu.sync_copy(out_v, out_hbm.at[pl.ds(base, BPW)])


def kernel(u_idx, i_idx, u_table, i_table):
    u_flat = jax.lax.optimization_barrier(
        u_table.T.reshape(N_ROWS * K // GRAN, GRAN))
    i_flat = jax.lax.optimization_barrier(
        i_table.T.reshape(N_ROWS * K // GRAN, GRAN))
    return _svd_dot(u_idx.astype(jnp.int32), i_idx.astype(jnp.int32),
                    u_flat, i_flat)


# speedup vs baseline: 3.6571x; 3.6571x over previous
import functools

import jax
import jax.numpy as jnp
from jax import lax
from jax.experimental import pallas as pl
from jax.experimental.pallas import tpu as pltpu
from jax.experimental.pallas import tpu_sc as plsc

N_U = 1000000
N_I = 1000000
K = 32
B = 16384

NC = 2
NS = 16
NW = NC * NS
BPW = B // NW
L = 16
G = BPW // L

_mesh = plsc.VectorSubcoreMesh(core_axis_name="c", subcore_axis_name="s")


@functools.partial(
    pl.kernel,
    out_type=jax.ShapeDtypeStruct((B,), jnp.float32),
    mesh=_mesh,
    scratch_types=[
        pltpu.VMEM((BPW,), jnp.int32),
        pltpu.VMEM((BPW,), jnp.int32),
        pltpu.VMEM((BPW, K), jnp.float32),
        pltpu.VMEM((BPW, K), jnp.float32),
        pltpu.VMEM((BPW,), jnp.float32),
        pltpu.SemaphoreType.DMA,
    ],
    compiler_params=pltpu.CompilerParams(needs_layout_passes=False,
                                         use_tc_tiling_on_sc=False),
)
def _svd_dot(u_idx_hbm, i_idx_hbm, u_table_hbm, i_table_hbm, out_hbm,
             uidx_v, iidx_v, urows_v, irows_v, out_v, sem):
    wid = lax.axis_index("s") * NC + lax.axis_index("c")
    base = wid * BPW

    pltpu.sync_copy(u_idx_hbm.at[pl.ds(base, BPW)], uidx_v)
    pltpu.sync_copy(i_idx_hbm.at[pl.ds(base, BPW)], iidx_v)

    cp_u = pltpu.async_copy(u_table_hbm.at[uidx_v], urows_v, sem)
    cp_i = pltpu.async_copy(i_table_hbm.at[iidx_v], irows_v, sem)
    cp_u.wait()
    cp_i.wait()

    lane = lax.iota(jnp.int32, 16)

    def group(g, carry):
        rows = g * L + lane
        acc = jnp.zeros((L,), jnp.float32)
        for k in range(K):
            col = jnp.full((L,), k, jnp.int32)
            uv = plsc.load_gather(urows_v, [rows, col])
            iv = plsc.load_gather(irows_v, [rows, col])
            acc = acc + uv * iv
        out_v[pl.ds(g * L, L)] = acc
        return carry

    lax.fori_loop(0, G, group, 0)

    pltpu.sync_copy(out_v, out_hbm.at[pl.ds(base, BPW)])


def kernel(u_idx, i_idx, u_table, i_table):
    return _svd_dot(u_idx.astype(jnp.int32), i_idx.astype(jnp.int32),
                    u_table, i_table)
